# baseline (speedup 1.0000x reference)
import jax
import jax.numpy as jnp
from jax.experimental import pallas as pl
from jax.experimental.pallas import tpu as pltpu

S = 2048
D = 768
NH = 12
HDIM = 64
E = 8
FF = 1536
EPS = 1e-06
TQ = 256
TT = 256
NT = 24
P = NT * TT


def _qkv_kernel(x_ref, wq_ref, wk_ref, wv_ref, b_ref, oq_ref, ok_ref,
                ov_ref):
    x = x_ref[...]
    b = b_ref[...]
    yq = jnp.dot(x, wq_ref[...], preferred_element_type=jnp.float32) \
        + b[:, :D]
    oq_ref[...] = (yq * (HDIM ** -0.5)).astype(jnp.bfloat16)
    yk = jnp.dot(x, wk_ref[...], preferred_element_type=jnp.float32) \
        + b[:, D:2 * D]
    ok_ref[...] = yk.astype(jnp.bfloat16)
    yv = jnp.dot(x, wv_ref[...], preferred_element_type=jnp.float32) \
        + b[:, 2 * D:]
    ov_ref[...] = yv.astype(jnp.bfloat16)


def _attn_kernel(q_ref, k_ref, v_ref, m_ref, o_ref):
    mask = m_ref[...].astype(jnp.bfloat16)
    for h in range(NH):
        sl = slice(h * HDIM, (h + 1) * HDIM)
        q = q_ref[:, sl]
        k = k_ref[:, sl]
        s = jax.lax.dot_general(q, k, (((1,), (1,)), ((), ())),
                                preferred_element_type=jnp.float32)
        sb = s.astype(jnp.bfloat16) + mask
        mx = jnp.max(sb, axis=-1, keepdims=True)
        p = jnp.exp(sb - mx)
        l = jnp.sum(p, axis=-1, keepdims=True, dtype=jnp.float32)
        o = jnp.dot(p, v_ref[:, sl], preferred_element_type=jnp.float32)
        o_ref[:, sl] = o / l


def _oproj_kernel(ao_ref, w_ref, b_ref, x_ref, g_ref, o_ref):
    y = jnp.dot(ao_ref[...], w_ref[...],
                preferred_element_type=jnp.float32) + b_ref[...]
    y = y + x_ref[...]
    var = jnp.mean(y * y, axis=-1, keepdims=True)
    o_ref[...] = g_ref[...] * (y * jax.lax.rsqrt(var + EPS))


def _router_kernel(h_ref, gw_ref, logits_ref, p0_ref, p1_ref, wts_ref,
                   et_ref, tv_ref):
    h = h_ref[...]
    logits = jnp.dot(h, gw_ref[...], preferred_element_type=jnp.float32)
    logits_ref[...] = logits
    m = jnp.max(logits, axis=-1, keepdims=True)
    ex = jnp.exp(logits - m)
    probs = ex / jnp.sum(ex, axis=-1, keepdims=True)
    lane = jax.lax.broadcasted_iota(jnp.int32, (S, E), 1)
    m0 = jnp.max(probs, axis=-1, keepdims=True)
    i0 = jnp.min(jnp.where(probs == m0, lane, E), axis=-1, keepdims=True)
    masked = jnp.where(lane == i0, -1.0, probs)
    m1 = jnp.max(masked, axis=-1, keepdims=True)
    i1 = jnp.min(jnp.where(masked == m1, lane, E), axis=-1, keepdims=True)
    den = m0 + m1 + 1e-20
    w0 = m0 / den / 2.0
    w1 = m1 / den / 2.0
    oh0 = (lane == i0).astype(jnp.float32)
    oh1 = (lane == i1).astype(jnp.float32)
    bmat = oh0 + oh1
    chunks = []
    csz = 256
    for ci in range(S // csz):
        r = jax.lax.broadcasted_iota(jnp.int32, (csz, S), 0) + ci * csz
        c = jax.lax.broadcasted_iota(jnp.int32, (csz, S), 1)
        ltri = (c < r).astype(jnp.bfloat16)
        chunks.append(jnp.dot(ltri, bmat.astype(jnp.bfloat16),
                              preferred_element_type=jnp.float32))
    cum = jnp.concatenate(chunks, axis=0)
    counts = jnp.sum(bmat, axis=0, keepdims=True)
    padded = jnp.ceil(counts / TT) * TT
    er = jax.lax.broadcasted_iota(jnp.int32, (E, E), 0)
    ec = jax.lax.broadcasted_iota(jnp.int32, (E, E), 1)
    moff = (er < ec).astype(jnp.float32)
    offs = jnp.dot(padded, moff, preferred_element_type=jnp.float32)
    ends = offs + padded
    posmat = offs + cum
    pos0 = jnp.sum(oh0 * posmat, axis=-1, keepdims=True)
    pos1 = jnp.sum(oh1 * posmat, axis=-1, keepdims=True)
    p0_ref[...] = pos0.astype(jnp.int32)
    p1_ref[...] = pos1.astype(jnp.int32)
    wts_ref[...] = jnp.concatenate([w0, w1], axis=1)
    tstart = jax.lax.broadcasted_iota(
        jnp.int32, (NT, E), 0).astype(jnp.float32) * TT
    etile = jnp.sum((jnp.broadcast_to(ends, (NT, E)) <= tstart)
                    .astype(jnp.int32), axis=-1, keepdims=True)
    ptot = jnp.sum(padded)
    valid = (tstart[:, 0:1] < ptot).astype(jnp.int32)
    elast = jnp.sum((ends <= ptot - TT).astype(jnp.int32))
    etile = jnp.where(valid > 0, etile, elast)
    et_ref[...] = etile
    tv_ref[...] = valid


def _scatter_kernel(p0_ref, p1_ref, h_ref, xs_ref):
    def body(t, carry):
        row = h_ref[pl.ds(t, 1), :]
        xs_ref[pl.ds(p0_ref[t], 1), :] = row
        xs_ref[pl.ds(p1_ref[t], 1), :] = row
        return carry
    jax.lax.fori_loop(0, S, body, 0, unroll=8)


def _ffn_kernel(e_ref, v_ref, xs_ref, w1_ref, b1_ref, w2_ref, b2_ref, o_ref):
    t = pl.program_id(0)

    @pl.when(v_ref[t] > 0)
    def _():
        x = xs_ref[...]
        hf = FF // 2
        w1 = w1_ref[0]
        w2 = w2_ref[0]
        b1 = b1_ref[0]
        ha = jnp.dot(x, w1[:, :hf], preferred_element_type=jnp.float32) \
            + b1[:, :hf]
        hb = jnp.dot(x, w1[:, hf:], preferred_element_type=jnp.float32) \
            + b1[:, hf:]
        ga = 0.5 * ha * (1.0 + jax.lax.erf(ha * (2.0 ** -0.5)))
        ya = jnp.dot(ga, w2[:hf, :], preferred_element_type=jnp.float32)
        gb = 0.5 * hb * (1.0 + jax.lax.erf(hb * (2.0 ** -0.5)))
        yb = jnp.dot(gb, w2[hf:, :], preferred_element_type=jnp.float32)
        o_ref[...] = ya + yb + b2_ref[0]


def _combine_kernel(p0_ref, p1_ref, ys_ref, h1_ref, w_ref, g_ref, b_ref,
                    o_ref, r0_ref, r1_ref):
    def body(t, carry):
        r0_ref[pl.ds(t, 1), :] = ys_ref[pl.ds(p0_ref[t], 1), :]
        r1_ref[pl.ds(t, 1), :] = ys_ref[pl.ds(p1_ref[t], 1), :]
        return carry
    jax.lax.fori_loop(0, S, body, 0, unroll=8)
    w = w_ref[...]
    y = h1_ref[...] + w[:, 0:1] * r0_ref[...] + w[:, 1:2] * r1_ref[...]
    mu = jnp.mean(y, axis=-1, keepdims=True)
    yc = y - mu
    var = jnp.mean(yc * yc, axis=-1, keepdims=True)
    o_ref[...] = yc * jax.lax.rsqrt(var + EPS) * g_ref[...] + b_ref[...]


def kernel(hidden_states, attn_mask, output_attentions, Wq, bq, Wk, bk,
           Wv, bv, Wo, bo, rms_w, gate_w, W1, b1, W2, b2, ln_w, ln_b):
    x = hidden_states.reshape(S, D).astype(jnp.float32)
    mask = attn_mask.reshape(S, S).astype(jnp.float32)

    bqkv = jnp.concatenate([bq, bk, bv]).reshape(1, 3 * D)
    qb, kb, vb = pl.pallas_call(
        _qkv_kernel,
        grid=(S // TQ,),
        in_specs=[pl.BlockSpec((TQ, D), lambda i: (i, 0)),
                  pl.BlockSpec((D, D), lambda i: (0, 0)),
                  pl.BlockSpec((D, D), lambda i: (0, 0)),
                  pl.BlockSpec((D, D), lambda i: (0, 0)),
                  pl.BlockSpec((1, 3 * D), lambda i: (0, 0))],
        out_specs=[pl.BlockSpec((TQ, D), lambda i: (i, 0))] * 3,
        out_shape=[jax.ShapeDtypeStruct((S, D), jnp.bfloat16)] * 3,
    )(x, Wq.T, Wk.T, Wv.T, bqkv)

    ao = pl.pallas_call(
        _attn_kernel,
        grid=(S // TQ,),
        in_specs=[pl.BlockSpec((TQ, D), lambda i: (i, 0)),
                  pl.BlockSpec((S, D), lambda i: (0, 0)),
                  pl.BlockSpec((S, D), lambda i: (0, 0)),
                  pl.BlockSpec((TQ, S), lambda i: (i, 0))],
        out_specs=pl.BlockSpec((TQ, D), lambda i: (i, 0)),
        out_shape=jax.ShapeDtypeStruct((S, D), jnp.float32),
    )(qb, kb, vb, mask)

    h1 = pl.pallas_call(
        _oproj_kernel,
        in_specs=[pl.BlockSpec((S, D), lambda: (0, 0)),
                  pl.BlockSpec((D, D), lambda: (0, 0)),
                  pl.BlockSpec((1, D), lambda: (0, 0)),
                  pl.BlockSpec((S, D), lambda: (0, 0)),
                  pl.BlockSpec((1, D), lambda: (0, 0))],
        out_specs=pl.BlockSpec((S, D), lambda: (0, 0)),
        out_shape=jax.ShapeDtypeStruct((S, D), jnp.float32),
    )(ao, Wo.T, bo.reshape(1, D), x, rms_w.reshape(1, D))

    logits, pos0, pos1, wts, etile, tvalid = pl.pallas_call(
        _router_kernel,
        in_specs=[pl.BlockSpec((S, D), lambda: (0, 0)),
                  pl.BlockSpec((D, E), lambda: (0, 0))],
        out_specs=[pl.BlockSpec((S, E), lambda: (0, 0)),
                   pl.BlockSpec((S, 1), lambda: (0, 0)),
                   pl.BlockSpec((S, 1), lambda: (0, 0)),
                   pl.BlockSpec((S, 2), lambda: (0, 0)),
                   pl.BlockSpec((NT, 1), lambda: (0, 0)),
                   pl.BlockSpec((NT, 1), lambda: (0, 0))],
        out_shape=[jax.ShapeDtypeStruct((S, E), jnp.float32),
                   jax.ShapeDtypeStruct((S, 1), jnp.int32),
                   jax.ShapeDtypeStruct((S, 1), jnp.int32),
                   jax.ShapeDtypeStruct((S, 2), jnp.float32),
                   jax.ShapeDtypeStruct((NT, 1), jnp.int32),
                   jax.ShapeDtypeStruct((NT, 1), jnp.int32)],
    )(h1, gate_w.T)
    pos0 = pos0.reshape(S)
    pos1 = pos1.reshape(S)
    etile = etile.reshape(NT)
    tvalid = tvalid.reshape(NT)

    xs = pl.pallas_call(
        _scatter_kernel,
        grid_spec=pltpu.PrefetchScalarGridSpec(
            num_scalar_prefetch=2,
            grid=(1,),
            in_specs=[pl.BlockSpec((S, D), lambda i, p0, p1: (0, 0))],
            out_specs=pl.BlockSpec((P, D), lambda i, p0, p1: (0, 0))),
        out_shape=jax.ShapeDtypeStruct((P, D), jnp.float32),
    )(pos0, pos1, h1)

    ys = pl.pallas_call(
        _ffn_kernel,
        grid_spec=pltpu.PrefetchScalarGridSpec(
            num_scalar_prefetch=2,
            grid=(NT,),
            in_specs=[pl.BlockSpec((TT, D), lambda t, e, v: (t, 0)),
                      pl.BlockSpec((1, D, FF),
                                   lambda t, e, v: (e[t], 0, 0)),
                      pl.BlockSpec((1, 1, FF),
                                   lambda t, e, v: (e[t], 0, 0)),
                      pl.BlockSpec((1, FF, D),
                                   lambda t, e, v: (e[t], 0, 0)),
                      pl.BlockSpec((1, 1, D),
                                   lambda t, e, v: (e[t], 0, 0))],
            out_specs=pl.BlockSpec((TT, D), lambda t, e, v: (t, 0))),
        out_shape=jax.ShapeDtypeStruct((P, D), jnp.float32),
    )(etile, tvalid, xs, W1, b1.reshape(E, 1, FF), W2, b2.reshape(E, 1, D))

    out = pl.pallas_call(
        _combine_kernel,
        grid_spec=pltpu.PrefetchScalarGridSpec(
            num_scalar_prefetch=2,
            grid=(1,),
            in_specs=[pl.BlockSpec((P, D), lambda i, p0, p1: (0, 0)),
                      pl.BlockSpec((S, D), lambda i, p0, p1: (0, 0)),
                      pl.BlockSpec((S, 2), lambda i, p0, p1: (0, 0)),
                      pl.BlockSpec((1, D), lambda i, p0, p1: (0, 0)),
                      pl.BlockSpec((1, D), lambda i, p0, p1: (0, 0))],
            out_specs=pl.BlockSpec((S, D), lambda i, p0, p1: (0, 0)),
            scratch_shapes=[pltpu.VMEM((S, D), jnp.float32),
                            pltpu.VMEM((S, D), jnp.float32)]),
        out_shape=jax.ShapeDtypeStruct((S, D), jnp.float32),
    )(pos0, pos1, ys, h1, wts, ln_w.reshape(1, D), ln_b.reshape(1, D))

    return out.reshape(1, S, D), logits


# speedup vs baseline: 1.0918x; 1.0918x over previous
import jax
import jax.numpy as jnp
from jax.experimental import pallas as pl
from jax.experimental.pallas import tpu as pltpu

S = 2048
D = 768
NH = 12
HDIM = 64
E = 8
FF = 1536
EPS = 1e-06
TQ = 256
TA = 512
TT = 256
NT = 24
P = NT * TT


def _qkv_kernel(x_ref, wq_ref, wk_ref, wv_ref, b_ref, oq_ref, ok_ref,
                ov_ref):
    x = x_ref[...]
    b = b_ref[...]
    yq = jnp.dot(x, wq_ref[...], preferred_element_type=jnp.float32) \
        + b[:, :D]
    oq_ref[...] = (yq * (HDIM ** -0.5)).astype(jnp.bfloat16)
    yk = jnp.dot(x, wk_ref[...], preferred_element_type=jnp.float32) \
        + b[:, D:2 * D]
    ok_ref[...] = yk.astype(jnp.bfloat16)
    yv = jnp.dot(x, wv_ref[...], preferred_element_type=jnp.float32) \
        + b[:, 2 * D:]
    ov_ref[...] = yv.astype(jnp.bfloat16)


def _attn_kernel(q_ref, k_ref, v_ref, m_ref, o_ref):
    mask = m_ref[...]
    for h in range(NH):
        sl = slice(h * HDIM, (h + 1) * HDIM)
        q = q_ref[:, sl]
        k = k_ref[:, sl]
        s = jax.lax.dot_general(q, k, (((1,), (1,)), ((), ())),
                                preferred_element_type=jnp.float32)
        sb = (s + mask).astype(jnp.bfloat16)
        mx = jnp.max(sb, axis=-1, keepdims=True)
        p = jnp.exp(sb - mx)
        l = jnp.sum(p, axis=-1, keepdims=True, dtype=jnp.float32)
        o = jnp.dot(p, v_ref[:, sl], preferred_element_type=jnp.float32)
        o_ref[:, sl] = o / l


def _oproj_kernel(ao_ref, w_ref, b_ref, x_ref, g_ref, o_ref):
    y = jnp.dot(ao_ref[...], w_ref[...],
                preferred_element_type=jnp.float32) + b_ref[...]
    y = y + x_ref[...]
    var = jnp.mean(y * y, axis=-1, keepdims=True)
    o_ref[...] = g_ref[...] * (y * jax.lax.rsqrt(var + EPS))


def _router_kernel(h_ref, gw_ref, logits_ref, p0_ref, p1_ref, wts_ref,
                   et_ref, tv_ref):
    h = h_ref[...]
    logits = jnp.dot(h, gw_ref[...], preferred_element_type=jnp.float32)
    logits_ref[...] = logits
    m = jnp.max(logits, axis=-1, keepdims=True)
    ex = jnp.exp(logits - m)
    probs = ex / jnp.sum(ex, axis=-1, keepdims=True)
    lane = jax.lax.broadcasted_iota(jnp.int32, (S, E), 1)
    m0 = jnp.max(probs, axis=-1, keepdims=True)
    i0 = jnp.min(jnp.where(probs == m0, lane, E), axis=-1, keepdims=True)
    masked = jnp.where(lane == i0, -1.0, probs)
    m1 = jnp.max(masked, axis=-1, keepdims=True)
    i1 = jnp.min(jnp.where(masked == m1, lane, E), axis=-1, keepdims=True)
    den = m0 + m1 + 1e-20
    w0 = m0 / den / 2.0
    w1 = m1 / den / 2.0
    oh0 = (lane == i0).astype(jnp.float32)
    oh1 = (lane == i1).astype(jnp.float32)
    bmat = oh0 + oh1
    chunks = []
    csz = 256
    for ci in range(S // csz):
        r = jax.lax.broadcasted_iota(jnp.int32, (csz, S), 0) + ci * csz
        c = jax.lax.broadcasted_iota(jnp.int32, (csz, S), 1)
        ltri = (c < r).astype(jnp.bfloat16)
        chunks.append(jnp.dot(ltri, bmat.astype(jnp.bfloat16),
                              preferred_element_type=jnp.float32))
    cum = jnp.concatenate(chunks, axis=0)
    counts = jnp.sum(bmat, axis=0, keepdims=True)
    padded = jnp.ceil(counts / TT) * TT
    er = jax.lax.broadcasted_iota(jnp.int32, (E, E), 0)
    ec = jax.lax.broadcasted_iota(jnp.int32, (E, E), 1)
    moff = (er < ec).astype(jnp.float32)
    offs = jnp.dot(padded, moff, preferred_element_type=jnp.float32)
    ends = offs + padded
    posmat = offs + cum
    pos0 = jnp.sum(oh0 * posmat, axis=-1, keepdims=True)
    pos1 = jnp.sum(oh1 * posmat, axis=-1, keepdims=True)
    p0_ref[...] = pos0.astype(jnp.int32)
    p1_ref[...] = pos1.astype(jnp.int32)
    wts_ref[...] = jnp.concatenate([w0, w1], axis=1)
    tstart = jax.lax.broadcasted_iota(
        jnp.int32, (NT, E), 0).astype(jnp.float32) * TT
    etile = jnp.sum((jnp.broadcast_to(ends, (NT, E)) <= tstart)
                    .astype(jnp.int32), axis=-1, keepdims=True)
    ptot = jnp.sum(padded)
    valid = (tstart[:, 0:1] < ptot).astype(jnp.int32)
    elast = jnp.sum((ends <= ptot - TT).astype(jnp.int32))
    etile = jnp.where(valid > 0, etile, elast)
    et_ref[...] = etile
    tv_ref[...] = valid


def _scatter_kernel(p0_ref, p1_ref, h_ref, xs_ref):
    def body(t, carry):
        row = h_ref[pl.ds(t, 1), :]
        xs_ref[pl.ds(p0_ref[t], 1), :] = row
        xs_ref[pl.ds(p1_ref[t], 1), :] = row
        return carry
    jax.lax.fori_loop(0, S, body, 0, unroll=8)


def _ffn_kernel(e_ref, v_ref, xs_ref, w1_ref, b1_ref, w2_ref, b2_ref, o_ref):
    t = pl.program_id(0)

    @pl.when(v_ref[t] > 0)
    def _():
        x = xs_ref[...]
        h = jnp.dot(x, w1_ref[0], preferred_element_type=jnp.float32) \
            + b1_ref[0]
        h = 0.5 * h * (1.0 + jax.lax.erf(h * (2.0 ** -0.5)))
        y = jnp.dot(h, w2_ref[0], preferred_element_type=jnp.float32) \
            + b2_ref[0]
        o_ref[...] = y


def _combine_kernel(p0_ref, p1_ref, ys_ref, h1_ref, w_ref, g_ref, b_ref,
                    o_ref, r0_ref, r1_ref):
    def body(t, carry):
        r0_ref[pl.ds(t, 1), :] = ys_ref[pl.ds(p0_ref[t], 1), :]
        r1_ref[pl.ds(t, 1), :] = ys_ref[pl.ds(p1_ref[t], 1), :]
        return carry
    jax.lax.fori_loop(0, S, body, 0, unroll=8)
    w = w_ref[...]
    y = h1_ref[...] + w[:, 0:1] * r0_ref[...] + w[:, 1:2] * r1_ref[...]
    mu = jnp.mean(y, axis=-1, keepdims=True)
    yc = y - mu
    var = jnp.mean(yc * yc, axis=-1, keepdims=True)
    o_ref[...] = yc * jax.lax.rsqrt(var + EPS) * g_ref[...] + b_ref[...]


def kernel(hidden_states, attn_mask, output_attentions, Wq, bq, Wk, bk,
           Wv, bv, Wo, bo, rms_w, gate_w, W1, b1, W2, b2, ln_w, ln_b):
    x = hidden_states.reshape(S, D).astype(jnp.float32)
    mask = attn_mask.reshape(S, S).astype(jnp.float32)

    bqkv = jnp.concatenate([bq, bk, bv]).reshape(1, 3 * D)
    qb, kb, vb = pl.pallas_call(
        _qkv_kernel,
        grid=(S // TQ,),
        in_specs=[pl.BlockSpec((TQ, D), lambda i: (i, 0)),
                  pl.BlockSpec((D, D), lambda i: (0, 0)),
                  pl.BlockSpec((D, D), lambda i: (0, 0)),
                  pl.BlockSpec((D, D), lambda i: (0, 0)),
                  pl.BlockSpec((1, 3 * D), lambda i: (0, 0))],
        out_specs=[pl.BlockSpec((TQ, D), lambda i: (i, 0))] * 3,
        out_shape=[jax.ShapeDtypeStruct((S, D), jnp.bfloat16)] * 3,
    )(x, Wq.T, Wk.T, Wv.T, bqkv)

    ao = pl.pallas_call(
        _attn_kernel,
        grid=(S // TA,),
        in_specs=[pl.BlockSpec((TA, D), lambda i: (i, 0)),
                  pl.BlockSpec((S, D), lambda i: (0, 0)),
                  pl.BlockSpec((S, D), lambda i: (0, 0)),
                  pl.BlockSpec((TA, S), lambda i: (i, 0))],
        out_specs=pl.BlockSpec((TA, D), lambda i: (i, 0)),
        out_shape=jax.ShapeDtypeStruct((S, D), jnp.float32),
    )(qb, kb, vb, mask)

    h1 = pl.pallas_call(
        _oproj_kernel,
        in_specs=[pl.BlockSpec((S, D), lambda: (0, 0)),
                  pl.BlockSpec((D, D), lambda: (0, 0)),
                  pl.BlockSpec((1, D), lambda: (0, 0)),
                  pl.BlockSpec((S, D), lambda: (0, 0)),
                  pl.BlockSpec((1, D), lambda: (0, 0))],
        out_specs=pl.BlockSpec((S, D), lambda: (0, 0)),
        out_shape=jax.ShapeDtypeStruct((S, D), jnp.float32),
    )(ao, Wo.T, bo.reshape(1, D), x, rms_w.reshape(1, D))

    logits, pos0, pos1, wts, etile, tvalid = pl.pallas_call(
        _router_kernel,
        in_specs=[pl.BlockSpec((S, D), lambda: (0, 0)),
                  pl.BlockSpec((D, E), lambda: (0, 0))],
        out_specs=[pl.BlockSpec((S, E), lambda: (0, 0)),
                   pl.BlockSpec((S, 1), lambda: (0, 0)),
                   pl.BlockSpec((S, 1), lambda: (0, 0)),
                   pl.BlockSpec((S, 2), lambda: (0, 0)),
                   pl.BlockSpec((NT, 1), lambda: (0, 0)),
                   pl.BlockSpec((NT, 1), lambda: (0, 0))],
        out_shape=[jax.ShapeDtypeStruct((S, E), jnp.float32),
                   jax.ShapeDtypeStruct((S, 1), jnp.int32),
                   jax.ShapeDtypeStruct((S, 1), jnp.int32),
                   jax.ShapeDtypeStruct((S, 2), jnp.float32),
                   jax.ShapeDtypeStruct((NT, 1), jnp.int32),
                   jax.ShapeDtypeStruct((NT, 1), jnp.int32)],
    )(h1, gate_w.T)
    pos0 = pos0.reshape(S)
    pos1 = pos1.reshape(S)
    etile = etile.reshape(NT)
    tvalid = tvalid.reshape(NT)

    xs = pl.pallas_call(
        _scatter_kernel,
        grid_spec=pltpu.PrefetchScalarGridSpec(
            num_scalar_prefetch=2,
            grid=(1,),
            in_specs=[pl.BlockSpec((S, D), lambda i, p0, p1: (0, 0))],
            out_specs=pl.BlockSpec((P, D), lambda i, p0, p1: (0, 0))),
        out_shape=jax.ShapeDtypeStruct((P, D), jnp.float32),
    )(pos0, pos1, h1)

    ys = pl.pallas_call(
        _ffn_kernel,
        grid_spec=pltpu.PrefetchScalarGridSpec(
            num_scalar_prefetch=2,
            grid=(NT,),
            in_specs=[pl.BlockSpec((TT, D), lambda t, e, v: (t, 0)),
                      pl.BlockSpec((1, D, FF),
                                   lambda t, e, v: (e[t], 0, 0)),
                      pl.BlockSpec((1, 1, FF),
                                   lambda t, e, v: (e[t], 0, 0)),
                      pl.BlockSpec((1, FF, D),
                                   lambda t, e, v: (e[t], 0, 0)),
                      pl.BlockSpec((1, 1, D),
                                   lambda t, e, v: (e[t], 0, 0))],
            out_specs=pl.BlockSpec((TT, D), lambda t, e, v: (t, 0))),
        out_shape=jax.ShapeDtypeStruct((P, D), jnp.float32),
    )(etile, tvalid, xs, W1, b1.reshape(E, 1, FF), W2, b2.reshape(E, 1, D))

    out = pl.pallas_call(
        _combine_kernel,
        grid_spec=pltpu.PrefetchScalarGridSpec(
            num_scalar_prefetch=2,
            grid=(1,),
            in_specs=[pl.BlockSpec((P, D), lambda i, p0, p1: (0, 0)),
                      pl.BlockSpec((S, D), lambda i, p0, p1: (0, 0)),
                      pl.BlockSpec((S, 2), lambda i, p0, p1: (0, 0)),
                      pl.BlockSpec((1, D), lambda i, p0, p1: (0, 0)),
                      pl.BlockSpec((1, D), lambda i, p0, p1: (0, 0))],
            out_specs=pl.BlockSpec((S, D), lambda i, p0, p1: (0, 0)),
            scratch_shapes=[pltpu.VMEM((S, D), jnp.float32),
                            pltpu.VMEM((S, D), jnp.float32)]),
        out_shape=jax.ShapeDtypeStruct((S, D), jnp.float32),
    )(pos0, pos1, ys, h1, wts, ln_w.reshape(1, D), ln_b.reshape(1, D))

    return out.reshape(1, S, D), logits
